# baseline (speedup 1.0000x reference)
import dataclasses
import functools

import jax
import jax.numpy as jnp
from jax import lax
from jax.experimental import pallas as pl
from jax.experimental.pallas import tpu as pltpu
from jax.experimental.pallas import tpu_sc as plsc

ROWS = 64
L = 1_000_000
THRESHOLD = 0.9

LP = 1_024_000
NTEC = 16
CH = LP // NTEC
HCH = CH // 2
NHIST = 4
SLEN = HCH // NHIST
SUB = 2_000
NSUB = SLEN // SUB
BINS = 2048
NBV = BINS // 16
DSL = BINS // NTEC

W8 = 12_800
WBLK = 8 * W8
NB = LP // WBLK

_NEG_INF = float("-inf")


def _iota2(shape):
  s = lax.broadcasted_iota(jnp.int32, shape, 0)
  c = lax.broadcasted_iota(jnp.int32, shape, 1)
  return s * shape[1] + c


def _block_cumsum(x):
  n = x.shape[1]
  lane = lax.broadcasted_iota(jnp.int32, x.shape, 1)
  k = 1
  while k < n:
    shifted = pltpu.roll(x, k, axis=1)
    x = x + jnp.where(lane >= k, shifted, 0.0)
    k *= 2
  row_tot = jnp.max(jnp.where(lane == n - 1, x, _NEG_INF), axis=1, keepdims=True)
  sub = lax.broadcasted_iota(jnp.int32, (8, 1), 0)
  kk = 1
  xoff = row_tot
  while kk < 8:
    sh = pltpu.roll(xoff, kk, axis=0)
    xoff = xoff + jnp.where(sub >= kk, sh, 0.0)
    kk *= 2
  return x + (xoff - row_tot)


def _update_best(v, x, io, carry, best):
  mv = jnp.max(v)

  @pl.when(mv > best[0])
  def _():
    cand = jnp.where(v == mv, io, jnp.int32(2**30))
    li = jnp.min(cand)
    x_sel = jnp.max(jnp.where(io == li, x, _NEG_INF))
    first = jnp.max(x)
    m_in = jnp.sum(jnp.where((x == x_sel) & (io < li), 1.0, 0.0))
    m_tot = m_in + jnp.where((first == x_sel) & (carry[1] == x_sel), carry[2], 0.0)
    best[0] = mv
    best[1] = x_sel
    best[2] = m_tot


def _scan_kernel(sv_ref, g_ref, outp_ref, outm_ref, carry, best):
  b = pl.program_id(1)

  @pl.when(b == 0)
  def _():
    carry[0] = 0.0
    carry[1] = -1.0
    carry[2] = 0.0
    best[0] = _NEG_INF
    best[1] = 0.0
    best[2] = 0.0

  x = sv_ref[0, 0]
  g = g_ref[0, 0]
  io = _iota2(x.shape)

  c0 = carry[0]
  s = jnp.sum(x)
  lv = jnp.min(x)
  fully_kept = (c0 + s) - lv < THRESHOLD
  fully_dropped = c0 >= THRESHOLD

  @pl.when(fully_kept)
  def _():
    v = jnp.log(x) + g
    _update_best(v, x, io, carry, best)

  @pl.when(jnp.logical_not(fully_kept) & jnp.logical_not(fully_dropped))
  def _():
    incl = _block_cumsum(x)
    excl = c0 + (incl - x)
    v = jnp.where(excl < THRESHOLD, jnp.log(x) + g, _NEG_INF)
    _update_best(v, x, io, carry, best)

  tail_in_block = jnp.sum(jnp.where(x == lv, 1.0, 0.0))
  whole_block = tail_in_block >= jnp.float32(WBLK)
  new_tail = jnp.where(whole_block & (lv == carry[1]),
                       tail_in_block + carry[2], tail_in_block)
  carry[1] = lv
  carry[2] = new_tail
  carry[0] = c0 + s

  outp_ref[0, 0, :] = jnp.full((128,), best[1], jnp.float32)
  outm_ref[0, 0, :] = jnp.full((128,), best[2], jnp.float32).astype(jnp.int32)


def _select_kernel(p_ref, ps_ref, m_ref, out_ref, carry):
  b = pl.program_id(1)

  @pl.when(b == 0)
  def _():
    carry[0] = 0
    carry[1] = -1

  p = p_ref[0, 0]
  pstar = jnp.max(ps_ref[0, 0, :])
  m = jnp.max(m_ref[0, 0, :])
  eq = p == pstar
  cnt = jnp.sum(eq.astype(jnp.int32))
  c = carry[0]

  @pl.when((m >= c) & (m < c + cnt))
  def _():
    eqf = eq.astype(jnp.float32)
    incl = _block_cumsum(eqf)
    rank_excl = (incl - eqf).astype(jnp.int32)
    io = _iota2(p.shape)
    sel = eq & (rank_excl == m - c)
    idx = jnp.max(jnp.where(sel, io, jnp.int32(-1)))
    carry[1] = idx + b * WBLK

  carry[0] = c + cnt
  out_ref[0, 0, :] = jnp.full((128,), carry[1], jnp.int32)


def _sc_sort(probs_p):
  mesh = plsc.VectorSubcoreMesh(
      core_axis_name="c", subcore_axis_name="s", num_cores=2, num_subcores=NTEC
  )
  cp = pltpu.CompilerParams()
  if "needs_layout_passes" in pltpu.CompilerParams.__dataclass_fields__:
    cp = dataclasses.replace(cp, needs_layout_passes=False)

  @functools.partial(
      pl.kernel,
      compiler_params=cp,
      out_type=[
          jax.ShapeDtypeStruct((ROWS * LP,), jnp.float32),
          jax.ShapeDtypeStruct((ROWS * LP,), jnp.float32),
      ],
      mesh=mesh,
      scratch_types=[
          pltpu.VMEM((HCH,), jnp.float32),
          pltpu.VMEM((SUB,), jnp.int32),
          pltpu.VMEM((SUB,), jnp.int32),
          pltpu.VMEM((SUB,), jnp.int32),
          pltpu.VMEM((SUB,), jnp.int32),
          pltpu.VMEM((BINS,), jnp.int32),
          pltpu.VMEM((BINS,), jnp.int32),
          pltpu.VMEM((BINS,), jnp.int32),
          pltpu.VMEM((BINS,), jnp.int32),
          pltpu.VMEM((BINS,), jnp.int32),
          pltpu.VMEM((BINS,), jnp.int32),
          pltpu.VMEM((BINS,), jnp.int32),
          pltpu.VMEM((BINS,), jnp.int32),
          pltpu.VMEM((BINS,), jnp.int32),
          pltpu.VMEM((BINS,), jnp.int32),
          pltpu.VMEM((BINS,), jnp.int32),
          pltpu.VMEM((NTEC, DSL), jnp.int32),
          pltpu.VMEM_SHARED((LP,), jnp.float32),
          pltpu.VMEM_SHARED((NTEC, BINS), jnp.int32),
          pltpu.SemaphoreType.DMA,
          pltpu.SemaphoreType.DMA,
      ],
  )
  def sortk(xin0, out1, out2, chunkbuf, posbuf0, posbuf1, posbuf2, posbuf3,
            hist0, hist1, hist2, hist3, curs, base, cntb, prefb, cursB, cursC,
            cursD, gblk, sprow, sgrid, sem0, sem1):
    hists = (hist0, hist1, hist2, hist3)
    scurs = (prefb, cursB, cursC, cursD)
    posbufs = (posbuf0, posbuf1, posbuf2, posbuf3)
    sems = (sem0, sem1)
    cid = lax.axis_index("c")
    sid = lax.axis_index("s")

    def digits_of(x, shift):
      bits = plsc.bitcast(x, jnp.int32)
      return (bits >> shift) & (BINS - 1)

    def one_pass(xin, xout, shift):
      @pl.loop(0, ROWS // 2)
      def _(i):
        row = cid + 2 * i
        rbase = row * LP

        @pl.loop(0, NBV)
        def _(v):
          z = jnp.zeros((16,), jnp.int32)
          for j in range(NHIST):
            hists[j][pl.ds(v * 16, 16)] = z

        for h in range(2):
          pltpu.sync_copy(
              xin.at[pl.ds(rbase + sid * CH + h * HCH, HCH)], chunkbuf)

          @pl.loop(0, HCH // (16 * NHIST))
          def _(g):
            for j in range(NHIST):
              d = digits_of(
                  chunkbuf[pl.ds(g * (16 * NHIST) + j * 16, 16)], shift)
              occ, lastm = plsc.scan_count(d)
              plsc.addupdate_scatter(hists[j], [d], occ, mask=lastm)

        @pl.loop(0, NBV)
        def _(v):
          sl = pl.ds(v * 16, 16)
          acc = hists[0][sl]
          for j in range(1, NHIST):
            acc = acc + hists[j][sl]
          hists[0][sl] = acc

        pltpu.sync_copy(hist0, sgrid.at[sid])
        plsc.subcore_barrier()

        grd = [
            pltpu.async_copy(
                sgrid.at[t, pl.ds(sid * DSL, DSL)], gblk.at[t], sem0)
            for t in range(NTEC)
        ]
        for h in grd:
          h.wait()
        for t in range(1, NTEC):
          @pl.loop(0, DSL // 16)
          def _(v, t=t):
            sl = pl.ds(v * 16, 16)
            gblk[t, sl] = gblk[t, sl] + gblk[t - 1, sl]
        grd = [
            pltpu.async_copy(
                gblk.at[t], sgrid.at[t, pl.ds(sid * DSL, DSL)], sem0)
            for t in range(NTEC)
        ]
        for h in grd:
          h.wait()
        plsc.subcore_barrier()

        pltpu.sync_copy(sgrid.at[sid], prefb)
        pltpu.sync_copy(sgrid.at[NTEC - 1], cntb)

        def suffix_body(j, carry):
          vi = NBV - 1 - j
          sl = pl.ds(vi * 16, 16)
          v = cntb[sl]
          incl = plsc.cumsum(v)
          s = jnp.sum(v)
          base[sl] = carry + (s - incl)
          return carry + s

        lax.fori_loop(0, NBV, suffix_body, jnp.int32(0))

        @pl.loop(0, NBV)
        def _(v):
          sl = pl.ds(v * 16, 16)
          curs[sl] = base[sl] + (prefb[sl] - hists[0][sl])

        for h in range(2):
          pltpu.sync_copy(
              xin.at[pl.ds(rbase + sid * CH + h * HCH, HCH)], chunkbuf)

          @pl.loop(0, NBV)
          def _(v):
            z = jnp.zeros((16,), jnp.int32)
            for j in range(NHIST):
              hists[j][pl.ds(v * 16, 16)] = z

          @pl.loop(0, SLEN // 16)
          def _(g):
            for j in range(NHIST):
              d = digits_of(chunkbuf[pl.ds(j * SLEN + g * 16, 16)], shift)
              occ, lastm = plsc.scan_count(d)
              plsc.addupdate_scatter(hists[j], [d], occ, mask=lastm)

          @pl.loop(0, NBV)
          def _(v):
            sl = pl.ds(v * 16, 16)
            a = curs[sl]
            b = a + hists[0][sl]
            c = b + hists[1][sl]
            dd = c + hists[2][sl]
            scurs[0][sl] = a
            scurs[1][sl] = b
            scurs[2][sl] = c
            scurs[3][sl] = dd
            curs[sl] = dd + hists[3][sl]

          @pl.loop(0, NSUB)
          def _(r):
            @pl.loop(0, SUB // 16)
            def _(g):
              for j in range(NHIST):
                d = digits_of(
                    chunkbuf[pl.ds(j * SLEN + r * SUB + g * 16, 16)], shift)
                occ, lastm = plsc.scan_count(d)
                b = plsc.load_gather(scurs[j], [d])
                posbufs[j][pl.ds(g * 16, 16)] = b + (occ - 1)
                plsc.addupdate_scatter(scurs[j], [d], occ, mask=lastm)

            handles = [
                pltpu.async_copy(
                    chunkbuf.at[pl.ds(j * SLEN + r * SUB, SUB)],
                    sprow.at[posbufs[j]], sems[j % 2])
                for j in range(NHIST)
            ]
            for hh in handles:
              hh.wait()

        plsc.subcore_barrier()

        pltpu.sync_copy(
            sprow.at[pl.ds(sid * CH, CH)],
            xout.at[pl.ds(rbase + sid * CH, CH)])

        plsc.subcore_barrier()

    one_pass(xin0, out1, 0)
    one_pass(out1, out2, 11)
    one_pass(out2, out1, 22)

  return sortk(probs_p.reshape(-1))[0].reshape(ROWS, LP)


def _scan_call(sv4, g4, interpret=False):
  return pl.pallas_call(
      _scan_kernel,
      grid=(ROWS, NB),
      in_specs=[
          pl.BlockSpec((1, 1, 8, W8), lambda r, b: (r, b, 0, 0)),
          pl.BlockSpec((1, 1, 8, W8), lambda r, b: (r, b, 0, 0)),
      ],
      out_specs=[
          pl.BlockSpec((1, 1, 128), lambda r, b: (r, 0, 0)),
          pl.BlockSpec((1, 1, 128), lambda r, b: (r, 0, 0)),
      ],
      out_shape=[
          jax.ShapeDtypeStruct((ROWS, 1, 128), jnp.float32),
          jax.ShapeDtypeStruct((ROWS, 1, 128), jnp.int32),
      ],
      scratch_shapes=[pltpu.SMEM((3,), jnp.float32), pltpu.SMEM((3,), jnp.float32)],
      interpret=interpret,
  )(sv4, g4)


def _select_call(p4, pstar, m, interpret=False):
  return pl.pallas_call(
      _select_kernel,
      grid=(ROWS, NB),
      in_specs=[
          pl.BlockSpec((1, 1, 8, W8), lambda r, b: (r, b, 0, 0)),
          pl.BlockSpec((1, 1, 128), lambda r, b: (r, 0, 0)),
          pl.BlockSpec((1, 1, 128), lambda r, b: (r, 0, 0)),
      ],
      out_specs=pl.BlockSpec((1, 1, 128), lambda r, b: (r, 0, 0)),
      out_shape=jax.ShapeDtypeStruct((ROWS, 1, 128), jnp.int32),
      scratch_shapes=[pltpu.SMEM((2,), jnp.int32)],
      interpret=interpret,
  )(p4, pstar, m)


@jax.jit
def _run(logits):
  probs = jax.nn.softmax(logits, axis=-1)
  noise = jax.random.gumbel(jax.random.key(42), (ROWS, L), jnp.float32)
  probs_p = jnp.pad(probs, ((0, 0), (0, LP - L)))
  noise_p = jnp.pad(noise, ((0, 0), (0, LP - L)))

  sv_p = _sc_sort(probs_p)

  sv4 = sv_p.reshape(ROWS, NB, 8, W8)
  g4 = noise_p.reshape(ROWS, NB, 8, W8)
  p4 = probs_p.reshape(ROWS, NB, 8, W8)

  pstar, m = _scan_call(sv4, g4)
  out = _select_call(p4, pstar, m)
  return out[:, 0, 0]


def kernel(logits):
  return _run(logits)


# speedup vs baseline: 1.4522x; 1.4522x over previous
import dataclasses
import functools

import jax
import jax.numpy as jnp
from jax import lax
from jax.experimental import pallas as pl
from jax.experimental.pallas import tpu as pltpu
from jax.experimental.pallas import tpu_sc as plsc

ROWS = 64
L = 1_000_000
THRESHOLD = 0.9

LP = 1_024_000
NTEC = 16
CH = LP // NTEC
HCH = CH // 2
NHIST = 4
WIN = 4_000
NWIN = HCH // WIN
BINS = 2048
NBV = BINS // 16
DSL = BINS // NTEC

W8 = 12_800
WBLK = 8 * W8
NB = LP // WBLK

_NEG_INF = float("-inf")


def _iota2(shape):
  s = lax.broadcasted_iota(jnp.int32, shape, 0)
  c = lax.broadcasted_iota(jnp.int32, shape, 1)
  return s * shape[1] + c


def _block_cumsum(x):
  n = x.shape[1]
  lane = lax.broadcasted_iota(jnp.int32, x.shape, 1)
  k = 1
  while k < n:
    shifted = pltpu.roll(x, k, axis=1)
    x = x + jnp.where(lane >= k, shifted, 0.0)
    k *= 2
  row_tot = jnp.max(jnp.where(lane == n - 1, x, _NEG_INF), axis=1, keepdims=True)
  sub = lax.broadcasted_iota(jnp.int32, (8, 1), 0)
  kk = 1
  xoff = row_tot
  while kk < 8:
    sh = pltpu.roll(xoff, kk, axis=0)
    xoff = xoff + jnp.where(sub >= kk, sh, 0.0)
    kk *= 2
  return x + (xoff - row_tot)


def _update_best(v, x, io, carry, best):
  mv = jnp.max(v)

  @pl.when(mv > best[0])
  def _():
    cand = jnp.where(v == mv, io, jnp.int32(2**30))
    li = jnp.min(cand)
    x_sel = jnp.max(jnp.where(io == li, x, _NEG_INF))
    first = jnp.max(x)
    m_in = jnp.sum(jnp.where((x == x_sel) & (io < li), 1.0, 0.0))
    m_tot = m_in + jnp.where((first == x_sel) & (carry[1] == x_sel), carry[2], 0.0)
    best[0] = mv
    best[1] = x_sel
    best[2] = m_tot


def _scan_kernel(sv_ref, g_ref, outp_ref, outm_ref, carry, best):
  b = pl.program_id(1)

  @pl.when(b == 0)
  def _():
    carry[0] = 0.0
    carry[1] = -1.0
    carry[2] = 0.0
    best[0] = _NEG_INF
    best[1] = 0.0
    best[2] = 0.0

  x = sv_ref[0, 0]
  g = g_ref[0, 0]
  io = _iota2(x.shape)

  c0 = carry[0]
  s = jnp.sum(x)
  lv = jnp.min(x)
  fully_kept = (c0 + s) - lv < THRESHOLD
  fully_dropped = c0 >= THRESHOLD

  @pl.when(fully_kept)
  def _():
    v = jnp.log(x) + g
    _update_best(v, x, io, carry, best)

  @pl.when(jnp.logical_not(fully_kept) & jnp.logical_not(fully_dropped))
  def _():
    incl = _block_cumsum(x)
    excl = c0 + (incl - x)
    v = jnp.where(excl < THRESHOLD, jnp.log(x) + g, _NEG_INF)
    _update_best(v, x, io, carry, best)

  tail_in_block = jnp.sum(jnp.where(x == lv, 1.0, 0.0))
  whole_block = tail_in_block >= jnp.float32(WBLK)
  new_tail = jnp.where(whole_block & (lv == carry[1]),
                       tail_in_block + carry[2], tail_in_block)
  carry[1] = lv
  carry[2] = new_tail
  carry[0] = c0 + s

  outp_ref[0, 0, :] = jnp.full((128,), best[1], jnp.float32)
  outm_ref[0, 0, :] = jnp.full((128,), best[2], jnp.float32).astype(jnp.int32)


def _select_kernel(p_ref, ps_ref, m_ref, out_ref, carry):
  b = pl.program_id(1)

  @pl.when(b == 0)
  def _():
    carry[0] = 0
    carry[1] = -1

  p = p_ref[0, 0]
  pstar = jnp.max(ps_ref[0, 0, :])
  m = jnp.max(m_ref[0, 0, :])
  eq = p == pstar
  cnt = jnp.sum(eq.astype(jnp.int32))
  c = carry[0]

  @pl.when((m >= c) & (m < c + cnt))
  def _():
    eqf = eq.astype(jnp.float32)
    incl = _block_cumsum(eqf)
    rank_excl = (incl - eqf).astype(jnp.int32)
    io = _iota2(p.shape)
    sel = eq & (rank_excl == m - c)
    idx = jnp.max(jnp.where(sel, io, jnp.int32(-1)))
    carry[1] = idx + b * WBLK

  carry[0] = c + cnt
  out_ref[0, 0, :] = jnp.full((128,), carry[1], jnp.int32)


def _sc_sort(probs_p):
  mesh = plsc.VectorSubcoreMesh(
      core_axis_name="c", subcore_axis_name="s", num_cores=2, num_subcores=NTEC
  )
  cp = pltpu.CompilerParams()
  if "needs_layout_passes" in pltpu.CompilerParams.__dataclass_fields__:
    cp = dataclasses.replace(cp, needs_layout_passes=False)

  @functools.partial(
      pl.kernel,
      compiler_params=cp,
      out_type=[
          jax.ShapeDtypeStruct((ROWS * LP,), jnp.float32),
          jax.ShapeDtypeStruct((ROWS * LP,), jnp.float32),
      ],
      mesh=mesh,
      scratch_types=[
          pltpu.VMEM((HCH,), jnp.float32),
          pltpu.VMEM((WIN,), jnp.int32),
          pltpu.VMEM((WIN,), jnp.int32),
          pltpu.VMEM((BINS,), jnp.int32),
          pltpu.VMEM((BINS,), jnp.int32),
          pltpu.VMEM((BINS,), jnp.int32),
          pltpu.VMEM((BINS,), jnp.int32),
          pltpu.VMEM((BINS,), jnp.int32),
          pltpu.VMEM((BINS,), jnp.int32),
          pltpu.VMEM((BINS,), jnp.int32),
          pltpu.VMEM((BINS,), jnp.int32),
          pltpu.VMEM((NTEC, DSL), jnp.int32),
          pltpu.VMEM_SHARED((LP,), jnp.float32),
          pltpu.VMEM_SHARED((NTEC, BINS), jnp.int32),
          pltpu.SemaphoreType.DMA,
          pltpu.SemaphoreType.DMA,
          pltpu.SemaphoreType.DMA,
      ],
  )
  def sortk(xin0, out1, out2, chunkbuf, posbuf0, posbuf1, hist0, hist1, hist2,
            hist3, curs, base, cntb, prefb, gblk, sprow, sgrid, sem0, sem1,
            semout):
    hists = (hist0, hist1, hist2, hist3)
    cid = lax.axis_index("c")
    sid = lax.axis_index("s")

    def digits_of(x, shift):
      bits = plsc.bitcast(x, jnp.int32)
      return (bits >> shift) & (BINS - 1)

    def one_pass(xin, xout, shift, pass_idx):
      @pl.loop(0, ROWS // 2)
      def _(i):
        row = cid + 2 * i
        rbase = row * LP

        @pl.loop(0, NBV)
        def _(v):
          z = jnp.zeros((16,), jnp.int32)
          for j in range(NHIST):
            hists[j][pl.ds(v * 16, 16)] = z

        for h in range(2):
          pltpu.sync_copy(
              xin.at[pl.ds(rbase + sid * CH + h * HCH, HCH)], chunkbuf)

          @pl.loop(0, HCH // (16 * NHIST))
          def _(g):
            for j in range(NHIST):
              d = digits_of(
                  chunkbuf[pl.ds(g * (16 * NHIST) + j * 16, 16)], shift)
              occ, lastm = plsc.scan_count(d)
              plsc.addupdate_scatter(hists[j], [d], occ, mask=lastm)

        @pl.loop(0, NBV)
        def _(v):
          sl = pl.ds(v * 16, 16)
          acc = hists[0][sl]
          for j in range(1, NHIST):
            acc = acc + hists[j][sl]
          hists[0][sl] = acc

        pltpu.sync_copy(hist0, sgrid.at[sid])
        plsc.subcore_barrier()

        grd = [
            pltpu.async_copy(
                sgrid.at[t, pl.ds(sid * DSL, DSL)], gblk.at[t], sem0)
            for t in range(NTEC)
        ]
        for h in grd:
          h.wait()
        for t in range(1, NTEC):
          @pl.loop(0, DSL // 16)
          def _(v, t=t):
            sl = pl.ds(v * 16, 16)
            gblk[t, sl] = gblk[t, sl] + gblk[t - 1, sl]
        grd = [
            pltpu.async_copy(
                gblk.at[t], sgrid.at[t, pl.ds(sid * DSL, DSL)], sem0)
            for t in range(NTEC)
        ]
        for h in grd:
          h.wait()
        plsc.subcore_barrier()

        pltpu.sync_copy(sgrid.at[sid], prefb)
        pltpu.sync_copy(sgrid.at[NTEC - 1], cntb)

        def suffix_body(j, carry):
          vi = NBV - 1 - j
          sl = pl.ds(vi * 16, 16)
          v = cntb[sl]
          incl = plsc.cumsum(v)
          s = jnp.sum(v)
          base[sl] = carry + (s - incl)
          return carry + s

        lax.fori_loop(0, NBV, suffix_body, jnp.int32(0))

        @pl.loop(0, NBV)
        def _(v):
          sl = pl.ds(v * 16, 16)
          curs[sl] = base[sl] + (prefb[sl] - hists[0][sl])

        def drain_out():
          pltpu.make_async_copy(
              sprow.at[pl.ds(sid * CH, CH)],
              xout.at[pl.ds(sid * CH, CH)], semout).wait()

        if pass_idx == 0:
          @pl.when(i > 0)
          def _():
            drain_out()
        else:
          drain_out()
        plsc.subcore_barrier()

        for h in range(2):
          pltpu.sync_copy(
              xin.at[pl.ds(rbase + sid * CH + h * HCH, HCH)], chunkbuf)

          @pl.loop(0, NWIN // 2)
          def _(w2):
            handles = []
            for k in range(2):
              w = 2 * w2 + k
              pb = (posbuf0, posbuf1)[k]
              sem = (sem0, sem1)[k]

              @pl.loop(0, WIN // 16)
              def _(v, w=w, pb=pb):
                sl = pl.ds(v * 16, 16)
                d = digits_of(chunkbuf[pl.ds(w * WIN + v * 16, 16)], shift)
                occ, lastm = plsc.scan_count(d)
                b = plsc.load_gather(curs, [d])
                pb[sl] = b + (occ - 1)
                plsc.addupdate_scatter(curs, [d], occ, mask=lastm)

              handles.append(pltpu.async_copy(
                  chunkbuf.at[pl.ds(w * WIN, WIN)], sprow.at[pb], sem))
            for hh in handles:
              hh.wait()

        plsc.subcore_barrier()

        pltpu.async_copy(
            sprow.at[pl.ds(sid * CH, CH)],
            xout.at[pl.ds(rbase + sid * CH, CH)], semout)

    one_pass(xin0, out1, 0, 0)
    one_pass(out1, out2, 11, 1)
    one_pass(out2, out1, 22, 2)
    pltpu.make_async_copy(
        sprow.at[pl.ds(sid * CH, CH)],
        out1.at[pl.ds(sid * CH, CH)], semout).wait()

  return sortk(probs_p.reshape(-1))[0].reshape(ROWS, LP)


def _scan_call(sv4, g4, interpret=False):
  return pl.pallas_call(
      _scan_kernel,
      grid=(ROWS, NB),
      in_specs=[
          pl.BlockSpec((1, 1, 8, W8), lambda r, b: (r, b, 0, 0)),
          pl.BlockSpec((1, 1, 8, W8), lambda r, b: (r, b, 0, 0)),
      ],
      out_specs=[
          pl.BlockSpec((1, 1, 128), lambda r, b: (r, 0, 0)),
          pl.BlockSpec((1, 1, 128), lambda r, b: (r, 0, 0)),
      ],
      out_shape=[
          jax.ShapeDtypeStruct((ROWS, 1, 128), jnp.float32),
          jax.ShapeDtypeStruct((ROWS, 1, 128), jnp.int32),
      ],
      scratch_shapes=[pltpu.SMEM((3,), jnp.float32), pltpu.SMEM((3,), jnp.float32)],
      interpret=interpret,
  )(sv4, g4)


def _select_call(p4, pstar, m, interpret=False):
  return pl.pallas_call(
      _select_kernel,
      grid=(ROWS, NB),
      in_specs=[
          pl.BlockSpec((1, 1, 8, W8), lambda r, b: (r, b, 0, 0)),
          pl.BlockSpec((1, 1, 128), lambda r, b: (r, 0, 0)),
          pl.BlockSpec((1, 1, 128), lambda r, b: (r, 0, 0)),
      ],
      out_specs=pl.BlockSpec((1, 1, 128), lambda r, b: (r, 0, 0)),
      out_shape=jax.ShapeDtypeStruct((ROWS, 1, 128), jnp.int32),
      scratch_shapes=[pltpu.SMEM((2,), jnp.int32)],
      interpret=interpret,
  )(p4, pstar, m)


@jax.jit
def _run(logits):
  probs = jax.nn.softmax(logits, axis=-1)
  noise = jax.random.gumbel(jax.random.key(42), (ROWS, L), jnp.float32)
  probs_p = jnp.pad(probs, ((0, 0), (0, LP - L)))
  noise_p = jnp.pad(noise, ((0, 0), (0, LP - L)))

  sv_p = _sc_sort(probs_p)

  sv4 = sv_p.reshape(ROWS, NB, 8, W8)
  g4 = noise_p.reshape(ROWS, NB, 8, W8)
  p4 = probs_p.reshape(ROWS, NB, 8, W8)

  pstar, m = _scan_call(sv4, g4)
  out = _select_call(p4, pstar, m)
  return out[:, 0, 0]


def kernel(logits):
  return _run(logits)
